# baseline (speedup 1.0000x reference)
import functools

import jax
import jax.numpy as jnp
from jax import lax
from jax.experimental import pallas as pl
from jax.experimental.pallas import tpu as pltpu
from jax.experimental.pallas import tpu_sc as plsc

N = 10000
E = 320000
D = 128

NC = 2
NS = 16
NW = NC * NS
G = 112
H = G // 2
CH = 6
CPW = 30
A_CH = 11
B_CH = CPW - A_CH
TOTCH = NS * CPW + 1
EP = TOTCH * CH * G

ROW_STRIDE = 624
ROW_SPAN = 640

_mesh = plsc.VectorSubcoreMesh(core_axis_name="c", subcore_axis_name="s")


@functools.partial(
    pl.kernel,
    out_type=jax.ShapeDtypeStruct((NC, N, D), jnp.float32),
    mesh=_mesh,
    scratch_types=[
        pltpu.VMEM((2, CH, G), jnp.int32),
        pltpu.VMEM((2, CH, G), jnp.int32),
        pltpu.VMEM((2, CH, G), jnp.float32),
        pltpu.VMEM((G, D), jnp.float32),
        pltpu.VMEM((G, D), jnp.float32),
        pltpu.VMEM((G, D), jnp.float32),
        pltpu.VMEM_SHARED((N, D), jnp.float32),
        pltpu.SemaphoreType.DMA,
        pltpu.SemaphoreType.DMA,
        pltpu.SemaphoreType.DMA,
        pltpu.SemaphoreType.DMA,
        pltpu.SemaphoreType.DMA,
        pltpu.SemaphoreType.DMA,
        pltpu.SemaphoreType.DMA,
        pltpu.SemaphoreType.DMA,
        pltpu.SemaphoreType.DMA,
        pltpu.SemaphoreType.DMA,
    ],
)
def _spmm_sc(pre_hbm, cols_hbm, rows_hbm, vals_hbm, out_hbm,
             colsb, rowsb, valsb, gb0, gb1, gb2, acc,
             sg0a, sg0b, sg1a, sg1b, sg2a, sg2b, ss0, ss1, ss2, si):
    c = lax.axis_index("c")
    s = lax.axis_index("s")
    wid = s * NC + c
    bufs = (gb0, gb1, gb2)
    semg = ((sg0a, sg0b), (sg1a, sg1b), (sg2a, sg2b))
    sems = (ss0, ss1, ss2)

    zero = jnp.zeros((16,), jnp.float32)

    def _zrow(r, carry):
        for j in range(D // 16):
            gb0[r, pl.ds(16 * j, 16)] = zero
        return carry

    lax.fori_loop(0, 80, _zrow, 0)
    for k in range(ROW_SPAN // 80):
        pltpu.sync_copy(
            gb0.at[pl.ds(0, 80)],
            acc.at[pl.ds(s * ROW_STRIDE + k * 80, 80)],
        )
    plsc.subcore_barrier()

    def _fetch_idx(chunk, slot):
        pltpu.async_copy(cols_hbm.at[chunk], colsb.at[slot], si)
        pltpu.async_copy(rows_hbm.at[chunk], rowsb.at[slot], si)
        pltpu.async_copy(vals_hbm.at[chunk], valsb.at[slot], si)

    def _wait_idx():
        pltpu.make_async_copy(cols_hbm.at[0], colsb.at[0], si).wait()
        pltpu.make_async_copy(rows_hbm.at[0], rowsb.at[0], si).wait()
        pltpu.make_async_copy(vals_hbm.at[0], valsb.at[0], si).wait()

    def _start_gather(buf, sempair, slot, pos):
        idx = colsb.at[slot, pos]
        pltpu.async_copy(pre_hbm.at[idx.at[pl.ds(0, H)]],
                         buf.at[pl.ds(0, H)], sempair[0])
        pltpu.async_copy(pre_hbm.at[idx.at[pl.ds(H, H)]],
                         buf.at[pl.ds(H, H)], sempair[1])

    def _wait_gather(buf, sempair):
        pltpu.make_async_copy(pre_hbm.at[pl.ds(0, H)],
                              buf.at[pl.ds(0, H)], sempair[0]).wait()
        pltpu.make_async_copy(pre_hbm.at[pl.ds(0, H)],
                              buf.at[pl.ds(H, H)], sempair[1]).wait()

    def _start_scatter(buf, sem, slot, pos):
        pltpu.async_copy(buf, acc.at[rowsb.at[slot, pos]], sem, add=True)

    def _wait_scatter(buf, sem):
        pltpu.make_async_copy(buf, acc.at[pl.ds(0, G)], sem).wait()

    def _scale(buf, slot, pos):
        def _eblock(eb, carry):
            vvec = valsb[slot, pos, pl.ds(eb * 16, 16)]
            for l in range(16):
                v = vvec[l]
                e = eb * 16 + l
                for j in range(D // 16):
                    sl = pl.ds(16 * j, 16)
                    buf[e, sl] = buf[e, sl] * v
            return carry

        lax.fori_loop(0, G // 16, _eblock, 0)

    def _body(g, k, slot, next_slot, first_chunk=False, last_chunk=False):
        k3 = k % 3
        j3 = (k + 1) % 3
        _wait_gather(bufs[k3], semg[k3])
        if not (first_chunk and k < 2):
            _wait_scatter(bufs[j3], sems[j3])
        if k == CH - 1:
            if not last_chunk:
                _wait_idx()
                _start_gather(bufs[j3], semg[j3], next_slot, 0)
        else:
            _start_gather(bufs[j3], semg[j3], slot, k + 1)
        _scale(bufs[k3], slot, k)
        _start_scatter(bufs[k3], sems[k3], slot, k)

    base = s * CPW + c * A_CH
    ncw = jnp.where(c == 0, A_CH, B_CH)

    _fetch_idx(base, 0)
    _wait_idx()
    _start_gather(gb0, semg[0], 0, 0)
    for k in range(2):
        _body(k, k, 0, 1, first_chunk=True)
    _fetch_idx(base + 1, 1)
    for k in range(2, CH):
        _body(k, k, 0, 1)

    def _super(ci, carry):
        slot = lax.rem(ci, 2)
        next_slot = lax.rem(ci + 1, 2)
        for k in range(2):
            _body(CH * ci + k, k, slot, next_slot)
        _fetch_idx(base + ci + 1, next_slot)
        for k in range(2, CH):
            _body(CH * ci + k, k, slot, next_slot)
        return carry

    lax.fori_loop(1, ncw, _super, 0)

    _wait_scatter(bufs[(CH - 2) % 3], sems[(CH - 2) % 3])
    _wait_scatter(bufs[(CH - 1) % 3], sems[(CH - 1) % 3])
    _wait_gather(bufs[0], semg[0])
    plsc.subcore_barrier()

    pltpu.sync_copy(
        acc.at[pl.ds(s * ROW_STRIDE, ROW_SPAN)],
        out_hbm.at[c, pl.ds(s * ROW_STRIDE, ROW_SPAN)],
    )


def _mm_body(x_ref, w_ref, o_ref):
    o_ref[...] = jnp.dot(x_ref[...], w_ref[...],
                         preferred_element_type=jnp.float32)


_matmul = pl.pallas_call(
    _mm_body,
    grid=(10,),
    in_specs=[
        pl.BlockSpec((N // 10, D), lambda i: (i, 0)),
        pl.BlockSpec((D, D), lambda i: (0, 0)),
    ],
    out_specs=pl.BlockSpec((N // 10, D), lambda i: (i, 0)),
    out_shape=jax.ShapeDtypeStruct((N, D), jnp.float32),
)


def _add_body(p_ref, o_ref):
    o_ref[...] = p_ref[0] + p_ref[1]


_add_partials = pl.pallas_call(
    _add_body,
    grid=(10,),
    in_specs=[pl.BlockSpec((NC, N // 10, D), lambda i: (0, i, 0))],
    out_specs=pl.BlockSpec((N // 10, D), lambda i: (i, 0)),
    out_shape=jax.ShapeDtypeStruct((N, D), jnp.float32),
)


def kernel(x, adj_indices, adj_values, W):
    pre_sup = _matmul(x, W)
    pad = EP - E
    cols = jnp.pad(adj_indices[1], (0, pad)).reshape(TOTCH, CH, G)
    rows = jnp.pad(adj_indices[0], (0, pad)).reshape(TOTCH, CH, G)
    vals = jnp.pad(adj_values, (0, pad)).reshape(TOTCH, CH, G)
    partials = _spmm_sc(pre_sup, cols, rows, vals)
    return _add_partials(partials)


# speedup vs baseline: 1.1780x; 1.1780x over previous
import functools

import jax
import jax.numpy as jnp
from jax import lax
from jax.experimental import pallas as pl
from jax.experimental.pallas import tpu as pltpu
from jax.experimental.pallas import tpu_sc as plsc

N = 10000
E = 320000
D = 128

NC = 2
NS = 16
NW = NC * NS
G = 112
H = G // 2
CH = 6
CPW = 30
A_CH = 11
B_CH = CPW - A_CH
TOTCH = NS * CPW + 1
EP = TOTCH * CH * G

ROW_STRIDE = 624
ROW_SPAN = 640

_mesh = plsc.VectorSubcoreMesh(core_axis_name="c", subcore_axis_name="s")


@functools.partial(
    pl.kernel,
    out_type=jax.ShapeDtypeStruct((NC, N, D), jnp.float32),
    mesh=_mesh,
    scratch_types=[
        pltpu.VMEM((2, CH, G), jnp.int32),
        pltpu.VMEM((2, CH, G), jnp.int32),
        pltpu.VMEM((2, CH, G), jnp.float32),
        pltpu.VMEM((G, D), jnp.float32),
        pltpu.VMEM((G, D), jnp.float32),
        pltpu.VMEM((G, D), jnp.float32),
        pltpu.VMEM_SHARED((N, D), jnp.float32),
        pltpu.SemaphoreType.DMA,
        pltpu.SemaphoreType.DMA,
        pltpu.SemaphoreType.DMA,
        pltpu.SemaphoreType.DMA,
        pltpu.SemaphoreType.DMA,
        pltpu.SemaphoreType.DMA,
        pltpu.SemaphoreType.DMA,
        pltpu.SemaphoreType.DMA,
        pltpu.SemaphoreType.DMA,
        pltpu.SemaphoreType.DMA,
    ],
)
def _spmm_sc(pre_hbm, cols_hbm, rows_hbm, vals_hbm, out_hbm,
             colsb, rowsb, valsb, gb0, gb1, gb2, acc,
             sg0a, sg0b, sg1a, sg1b, sg2a, sg2b, ss0, ss1, ss2, si):
    c = lax.axis_index("c")
    s = lax.axis_index("s")
    wid = s * NC + c
    bufs = (gb0, gb1, gb2)
    semg = ((sg0a, sg0b), (sg1a, sg1b), (sg2a, sg2b))
    sems = (ss0, ss1, ss2)

    zero = jnp.zeros((16,), jnp.float32)

    def _zrow(r, carry):
        for j in range(D // 16):
            gb0[r, pl.ds(16 * j, 16)] = zero
        return carry

    lax.fori_loop(0, 80, _zrow, 0)
    for k in range(ROW_SPAN // 80):
        pltpu.sync_copy(
            gb0.at[pl.ds(0, 80)],
            acc.at[pl.ds(s * ROW_STRIDE + k * 80, 80)],
        )
    plsc.subcore_barrier()

    def _fetch_idx(chunk, slot):
        pltpu.async_copy(cols_hbm.at[chunk], colsb.at[slot], si)
        pltpu.async_copy(rows_hbm.at[chunk], rowsb.at[slot], si)
        pltpu.async_copy(vals_hbm.at[chunk], valsb.at[slot], si)

    def _wait_idx():
        pltpu.make_async_copy(cols_hbm.at[0], colsb.at[0], si).wait()
        pltpu.make_async_copy(rows_hbm.at[0], rowsb.at[0], si).wait()
        pltpu.make_async_copy(vals_hbm.at[0], valsb.at[0], si).wait()

    def _start_gather(buf, sempair, slot, pos):
        idx = colsb.at[slot, pos]
        pltpu.async_copy(pre_hbm.at[idx.at[pl.ds(0, H)]],
                         buf.at[pl.ds(0, H)], sempair[0])
        pltpu.async_copy(pre_hbm.at[idx.at[pl.ds(H, H)]],
                         buf.at[pl.ds(H, H)], sempair[1])

    def _wait_gather(buf, sempair):
        pltpu.make_async_copy(pre_hbm.at[pl.ds(0, H)],
                              buf.at[pl.ds(0, H)], sempair[0]).wait()
        pltpu.make_async_copy(pre_hbm.at[pl.ds(0, H)],
                              buf.at[pl.ds(H, H)], sempair[1]).wait()

    def _start_scatter(buf, sem, slot, pos):
        pltpu.async_copy(buf, acc.at[rowsb.at[slot, pos]], sem, add=True)

    def _wait_scatter(buf, sem):
        pltpu.make_async_copy(buf, acc.at[pl.ds(0, G)], sem).wait()

    def _scale(buf, slot, pos):
        def _eblock(eb, carry):
            vvec = valsb[slot, pos, pl.ds(eb * 16, 16)]
            for l in range(16):
                v = vvec[l]
                e = eb * 16 + l
                for j in range(D // 16):
                    sl = pl.ds(16 * j, 16)
                    buf[e, sl] = buf[e, sl] * v
            return carry

        lax.fori_loop(0, G // 16, _eblock, 0)

    def _body(g, k, slot, next_slot, first_chunk=False, last_chunk=False):
        k3 = k % 3
        j3 = (k + 1) % 3
        _wait_gather(bufs[k3], semg[k3])
        if not (first_chunk and k < 2):
            _wait_scatter(bufs[j3], sems[j3])
        if k == CH - 1:
            if not last_chunk:
                _wait_idx()
                _start_gather(bufs[j3], semg[j3], next_slot, 0)
        else:
            _start_gather(bufs[j3], semg[j3], slot, k + 1)
        _scale(bufs[k3], slot, k)
        _start_scatter(bufs[k3], sems[k3], slot, k)

    base = s * CPW + c * B_CH
    ncw = jnp.where(c == 0, B_CH, A_CH)

    _fetch_idx(base, 0)
    _wait_idx()
    _start_gather(gb0, semg[0], 0, 0)
    for k in range(2):
        _body(k, k, 0, 1, first_chunk=True)
    _fetch_idx(base + 1, 1)
    for k in range(2, CH):
        _body(k, k, 0, 1)

    def _super(ci, carry):
        slot = lax.rem(ci, 2)
        next_slot = lax.rem(ci + 1, 2)
        for k in range(2):
            _body(CH * ci + k, k, slot, next_slot)
        _fetch_idx(base + ci + 1, next_slot)
        for k in range(2, CH):
            _body(CH * ci + k, k, slot, next_slot)
        return carry

    lax.fori_loop(1, ncw, _super, 0)

    _wait_scatter(bufs[(CH - 2) % 3], sems[(CH - 2) % 3])
    _wait_scatter(bufs[(CH - 1) % 3], sems[(CH - 1) % 3])
    _wait_gather(bufs[0], semg[0])
    plsc.subcore_barrier()

    pltpu.sync_copy(
        acc.at[pl.ds(s * ROW_STRIDE, ROW_SPAN)],
        out_hbm.at[c, pl.ds(s * ROW_STRIDE, ROW_SPAN)],
    )


def _mm_body(x_ref, w_ref, o_ref):
    o_ref[...] = jnp.dot(x_ref[...], w_ref[...],
                         preferred_element_type=jnp.float32)


_matmul = pl.pallas_call(
    _mm_body,
    grid=(10,),
    in_specs=[
        pl.BlockSpec((N // 10, D), lambda i: (i, 0)),
        pl.BlockSpec((D, D), lambda i: (0, 0)),
    ],
    out_specs=pl.BlockSpec((N // 10, D), lambda i: (i, 0)),
    out_shape=jax.ShapeDtypeStruct((N, D), jnp.float32),
)


def _add_body(p_ref, o_ref):
    o_ref[...] = p_ref[0] + p_ref[1]


_add_partials = pl.pallas_call(
    _add_body,
    grid=(10,),
    in_specs=[pl.BlockSpec((NC, N // 10, D), lambda i: (0, i, 0))],
    out_specs=pl.BlockSpec((N // 10, D), lambda i: (i, 0)),
    out_shape=jax.ShapeDtypeStruct((N, D), jnp.float32),
)


def kernel(x, adj_indices, adj_values, W):
    pre_sup = _matmul(x, W)
    pad = EP - E
    cols = jnp.pad(adj_indices[1], (0, pad)).reshape(TOTCH, CH, G)
    rows = jnp.pad(adj_indices[0], (0, pad)).reshape(TOTCH, CH, G)
    vals = jnp.pad(adj_values, (0, pad)).reshape(TOTCH, CH, G)
    partials = _spmm_sc(pre_sup, cols, rows, vals)
    return _add_partials(partials)


# speedup vs baseline: 1.2128x; 1.0296x over previous
import functools

import jax
import jax.numpy as jnp
from jax import lax
from jax.experimental import pallas as pl
from jax.experimental.pallas import tpu as pltpu
from jax.experimental.pallas import tpu_sc as plsc

N = 10000
E = 320000
D = 128

NC = 2
NS = 16
NW = NC * NS
G = 112
H = G // 2
CH = 6
CPW = 30
A_CH = 10
B_CH = CPW - A_CH
TOTCH = NS * CPW + 1
EP = TOTCH * CH * G

ROW_STRIDE = 624
ROW_SPAN = 640

_mesh = plsc.VectorSubcoreMesh(core_axis_name="c", subcore_axis_name="s")


@functools.partial(
    pl.kernel,
    out_type=jax.ShapeDtypeStruct((NC, N, D), jnp.float32),
    mesh=_mesh,
    scratch_types=[
        pltpu.VMEM((2, CH, G), jnp.int32),
        pltpu.VMEM((2, CH, G), jnp.int32),
        pltpu.VMEM((2, CH, G), jnp.float32),
        pltpu.VMEM((G, D), jnp.float32),
        pltpu.VMEM((G, D), jnp.float32),
        pltpu.VMEM((G, D), jnp.float32),
        pltpu.VMEM_SHARED((N, D), jnp.float32),
        pltpu.SemaphoreType.DMA,
        pltpu.SemaphoreType.DMA,
        pltpu.SemaphoreType.DMA,
        pltpu.SemaphoreType.DMA,
        pltpu.SemaphoreType.DMA,
        pltpu.SemaphoreType.DMA,
        pltpu.SemaphoreType.DMA,
        pltpu.SemaphoreType.DMA,
        pltpu.SemaphoreType.DMA,
        pltpu.SemaphoreType.DMA,
    ],
)
def _spmm_sc(pre_hbm, cols_hbm, rows_hbm, vals_hbm, out_hbm,
             colsb, rowsb, valsb, gb0, gb1, gb2, acc,
             sg0a, sg0b, sg1a, sg1b, sg2a, sg2b, ss0, ss1, ss2, si):
    c = lax.axis_index("c")
    s = lax.axis_index("s")
    wid = s * NC + c
    bufs = (gb0, gb1, gb2)
    semg = ((sg0a, sg0b), (sg1a, sg1b), (sg2a, sg2b))
    sems = (ss0, ss1, ss2)

    zero = jnp.zeros((16,), jnp.float32)

    def _zrow(r, carry):
        for j in range(D // 16):
            gb0[r, pl.ds(16 * j, 16)] = zero
        return carry

    lax.fori_loop(0, 80, _zrow, 0)
    for k in range(ROW_SPAN // 80):
        pltpu.sync_copy(
            gb0.at[pl.ds(0, 80)],
            acc.at[pl.ds(s * ROW_STRIDE + k * 80, 80)],
        )
    plsc.subcore_barrier()

    def _fetch_idx(chunk, slot):
        pltpu.async_copy(cols_hbm.at[chunk], colsb.at[slot], si)
        pltpu.async_copy(rows_hbm.at[chunk], rowsb.at[slot], si)
        pltpu.async_copy(vals_hbm.at[chunk], valsb.at[slot], si)

    def _wait_idx():
        pltpu.make_async_copy(cols_hbm.at[0], colsb.at[0], si).wait()
        pltpu.make_async_copy(rows_hbm.at[0], rowsb.at[0], si).wait()
        pltpu.make_async_copy(vals_hbm.at[0], valsb.at[0], si).wait()

    def _start_gather(buf, sempair, slot, pos):
        idx = colsb.at[slot, pos]
        pltpu.async_copy(pre_hbm.at[idx.at[pl.ds(0, H)]],
                         buf.at[pl.ds(0, H)], sempair[0])
        pltpu.async_copy(pre_hbm.at[idx.at[pl.ds(H, H)]],
                         buf.at[pl.ds(H, H)], sempair[1])

    def _wait_gather(buf, sempair):
        pltpu.make_async_copy(pre_hbm.at[pl.ds(0, H)],
                              buf.at[pl.ds(0, H)], sempair[0]).wait()
        pltpu.make_async_copy(pre_hbm.at[pl.ds(0, H)],
                              buf.at[pl.ds(H, H)], sempair[1]).wait()

    def _start_scatter(buf, sem, slot, pos):
        pltpu.async_copy(buf, acc.at[rowsb.at[slot, pos]], sem, add=True)

    def _wait_scatter(buf, sem):
        pltpu.make_async_copy(buf, acc.at[pl.ds(0, G)], sem).wait()

    def _scale(buf, slot, pos):
        def _eblock(eb, carry):
            vvec = valsb[slot, pos, pl.ds(eb * 16, 16)]
            for l in range(16):
                v = vvec[l]
                e = eb * 16 + l
                for j in range(D // 16):
                    sl = pl.ds(16 * j, 16)
                    buf[e, sl] = buf[e, sl] * v
            return carry

        lax.fori_loop(0, G // 16, _eblock, 0)

    def _body(g, k, slot, next_slot, first_chunk=False, last_chunk=False):
        k3 = k % 3
        j3 = (k + 1) % 3
        _wait_gather(bufs[k3], semg[k3])
        if not (first_chunk and k < 2):
            _wait_scatter(bufs[j3], sems[j3])
        if k == CH - 1:
            if not last_chunk:
                _wait_idx()
                _start_gather(bufs[j3], semg[j3], next_slot, 0)
        else:
            _start_gather(bufs[j3], semg[j3], slot, k + 1)
        _scale(bufs[k3], slot, k)
        _start_scatter(bufs[k3], sems[k3], slot, k)

    base = s * CPW + c * B_CH
    ncw = jnp.where(c == 0, B_CH, A_CH)

    _fetch_idx(base, 0)
    _wait_idx()
    _start_gather(gb0, semg[0], 0, 0)
    for k in range(2):
        _body(k, k, 0, 1, first_chunk=True)
    _fetch_idx(base + 1, 1)
    for k in range(2, CH):
        _body(k, k, 0, 1)

    def _super(ci, carry):
        slot = lax.rem(ci, 2)
        next_slot = lax.rem(ci + 1, 2)
        for k in range(2):
            _body(CH * ci + k, k, slot, next_slot)
        _fetch_idx(base + ci + 1, next_slot)
        for k in range(2, CH):
            _body(CH * ci + k, k, slot, next_slot)
        return carry

    lax.fori_loop(1, ncw, _super, 0)

    _wait_scatter(bufs[(CH - 2) % 3], sems[(CH - 2) % 3])
    _wait_scatter(bufs[(CH - 1) % 3], sems[(CH - 1) % 3])
    _wait_gather(bufs[0], semg[0])
    plsc.subcore_barrier()

    pltpu.sync_copy(
        acc.at[pl.ds(s * ROW_STRIDE, ROW_SPAN)],
        out_hbm.at[c, pl.ds(s * ROW_STRIDE, ROW_SPAN)],
    )


def _mm_body(x_ref, w_ref, o_ref):
    o_ref[...] = jnp.dot(x_ref[...], w_ref[...],
                         preferred_element_type=jnp.float32)


_matmul = pl.pallas_call(
    _mm_body,
    grid=(10,),
    in_specs=[
        pl.BlockSpec((N // 10, D), lambda i: (i, 0)),
        pl.BlockSpec((D, D), lambda i: (0, 0)),
    ],
    out_specs=pl.BlockSpec((N // 10, D), lambda i: (i, 0)),
    out_shape=jax.ShapeDtypeStruct((N, D), jnp.float32),
)


def _add_body(p_ref, o_ref):
    o_ref[...] = p_ref[0] + p_ref[1]


_add_partials = pl.pallas_call(
    _add_body,
    grid=(10,),
    in_specs=[pl.BlockSpec((NC, N // 10, D), lambda i: (0, i, 0))],
    out_specs=pl.BlockSpec((N // 10, D), lambda i: (i, 0)),
    out_shape=jax.ShapeDtypeStruct((N, D), jnp.float32),
)


def kernel(x, adj_indices, adj_values, W):
    pre_sup = _matmul(x, W)
    pad = EP - E
    cols = jnp.pad(adj_indices[1], (0, pad)).reshape(TOTCH, CH, G)
    rows = jnp.pad(adj_indices[0], (0, pad)).reshape(TOTCH, CH, G)
    vals = jnp.pad(adj_values, (0, pad)).reshape(TOTCH, CH, G)
    partials = _spmm_sc(pre_sup, cols, rows, vals)
    return _add_partials(partials)


# speedup vs baseline: 1.2315x; 1.0154x over previous
import functools

import jax
import jax.numpy as jnp
from jax import lax
from jax.experimental import pallas as pl
from jax.experimental.pallas import tpu as pltpu
from jax.experimental.pallas import tpu_sc as plsc

N = 10000
E = 320000
D = 128

NC = 2
NS = 16
NW = NC * NS
G = 112
H = G // 2
CH = 6
CPW = 30
A_CH = 9
B_CH = CPW - A_CH
TOTCH = NS * CPW + 1
EP = TOTCH * CH * G

ROW_STRIDE = 624
ROW_SPAN = 640

_mesh = plsc.VectorSubcoreMesh(core_axis_name="c", subcore_axis_name="s")


@functools.partial(
    pl.kernel,
    out_type=jax.ShapeDtypeStruct((NC, N, D), jnp.float32),
    mesh=_mesh,
    scratch_types=[
        pltpu.VMEM((2, CH, G), jnp.int32),
        pltpu.VMEM((2, CH, G), jnp.int32),
        pltpu.VMEM((2, CH, G), jnp.float32),
        pltpu.VMEM((G, D), jnp.float32),
        pltpu.VMEM((G, D), jnp.float32),
        pltpu.VMEM((G, D), jnp.float32),
        pltpu.VMEM_SHARED((N, D), jnp.float32),
        pltpu.SemaphoreType.DMA,
        pltpu.SemaphoreType.DMA,
        pltpu.SemaphoreType.DMA,
        pltpu.SemaphoreType.DMA,
        pltpu.SemaphoreType.DMA,
        pltpu.SemaphoreType.DMA,
        pltpu.SemaphoreType.DMA,
        pltpu.SemaphoreType.DMA,
        pltpu.SemaphoreType.DMA,
        pltpu.SemaphoreType.DMA,
    ],
)
def _spmm_sc(pre_hbm, cols_hbm, rows_hbm, vals_hbm, out_hbm,
             colsb, rowsb, valsb, gb0, gb1, gb2, acc,
             sg0a, sg0b, sg1a, sg1b, sg2a, sg2b, ss0, ss1, ss2, si):
    c = lax.axis_index("c")
    s = lax.axis_index("s")
    wid = s * NC + c
    bufs = (gb0, gb1, gb2)
    semg = ((sg0a, sg0b), (sg1a, sg1b), (sg2a, sg2b))
    sems = (ss0, ss1, ss2)

    zero = jnp.zeros((16,), jnp.float32)

    def _zrow(r, carry):
        for j in range(D // 16):
            gb0[r, pl.ds(16 * j, 16)] = zero
        return carry

    lax.fori_loop(0, 80, _zrow, 0)
    for k in range(ROW_SPAN // 80):
        pltpu.sync_copy(
            gb0.at[pl.ds(0, 80)],
            acc.at[pl.ds(s * ROW_STRIDE + k * 80, 80)],
        )
    plsc.subcore_barrier()

    def _fetch_idx(chunk, slot):
        pltpu.async_copy(cols_hbm.at[chunk], colsb.at[slot], si)
        pltpu.async_copy(rows_hbm.at[chunk], rowsb.at[slot], si)
        pltpu.async_copy(vals_hbm.at[chunk], valsb.at[slot], si)

    def _wait_idx():
        pltpu.make_async_copy(cols_hbm.at[0], colsb.at[0], si).wait()
        pltpu.make_async_copy(rows_hbm.at[0], rowsb.at[0], si).wait()
        pltpu.make_async_copy(vals_hbm.at[0], valsb.at[0], si).wait()

    def _start_gather(buf, sempair, slot, pos):
        idx = colsb.at[slot, pos]
        pltpu.async_copy(pre_hbm.at[idx.at[pl.ds(0, H)]],
                         buf.at[pl.ds(0, H)], sempair[0])
        pltpu.async_copy(pre_hbm.at[idx.at[pl.ds(H, H)]],
                         buf.at[pl.ds(H, H)], sempair[1])

    def _wait_gather(buf, sempair):
        pltpu.make_async_copy(pre_hbm.at[pl.ds(0, H)],
                              buf.at[pl.ds(0, H)], sempair[0]).wait()
        pltpu.make_async_copy(pre_hbm.at[pl.ds(0, H)],
                              buf.at[pl.ds(H, H)], sempair[1]).wait()

    def _start_scatter(buf, sem, slot, pos):
        pltpu.async_copy(buf, acc.at[rowsb.at[slot, pos]], sem, add=True)

    def _wait_scatter(buf, sem):
        pltpu.make_async_copy(buf, acc.at[pl.ds(0, G)], sem).wait()

    def _scale(buf, slot, pos):
        def _eblock(eb, carry):
            vvec = valsb[slot, pos, pl.ds(eb * 16, 16)]
            for l in range(16):
                v = vvec[l]
                e = eb * 16 + l
                for j in range(D // 16):
                    sl = pl.ds(16 * j, 16)
                    buf[e, sl] = buf[e, sl] * v
            return carry

        lax.fori_loop(0, G // 16, _eblock, 0)

    def _body(g, k, slot, next_slot, first_chunk=False, last_chunk=False):
        k3 = k % 3
        j3 = (k + 1) % 3
        _wait_gather(bufs[k3], semg[k3])
        if not (first_chunk and k < 2):
            _wait_scatter(bufs[j3], sems[j3])
        if k == CH - 1:
            if not last_chunk:
                _wait_idx()
                _start_gather(bufs[j3], semg[j3], next_slot, 0)
        else:
            _start_gather(bufs[j3], semg[j3], slot, k + 1)
        _scale(bufs[k3], slot, k)
        _start_scatter(bufs[k3], sems[k3], slot, k)

    base = s * CPW + c * B_CH
    ncw = jnp.where(c == 0, B_CH, A_CH)

    _fetch_idx(base, 0)
    _wait_idx()
    _start_gather(gb0, semg[0], 0, 0)
    for k in range(2):
        _body(k, k, 0, 1, first_chunk=True)
    _fetch_idx(base + 1, 1)
    for k in range(2, CH):
        _body(k, k, 0, 1)

    def _super(ci, carry):
        slot = lax.rem(ci, 2)
        next_slot = lax.rem(ci + 1, 2)
        for k in range(2):
            _body(CH * ci + k, k, slot, next_slot)
        _fetch_idx(base + ci + 1, next_slot)
        for k in range(2, CH):
            _body(CH * ci + k, k, slot, next_slot)
        return carry

    lax.fori_loop(1, ncw, _super, 0)

    _wait_scatter(bufs[(CH - 2) % 3], sems[(CH - 2) % 3])
    _wait_scatter(bufs[(CH - 1) % 3], sems[(CH - 1) % 3])
    _wait_gather(bufs[0], semg[0])
    plsc.subcore_barrier()

    pltpu.sync_copy(
        acc.at[pl.ds(s * ROW_STRIDE, ROW_SPAN)],
        out_hbm.at[c, pl.ds(s * ROW_STRIDE, ROW_SPAN)],
    )


def _mm_body(x_ref, w_ref, o_ref):
    o_ref[...] = jnp.dot(x_ref[...], w_ref[...],
                         preferred_element_type=jnp.float32)


_matmul = pl.pallas_call(
    _mm_body,
    grid=(10,),
    in_specs=[
        pl.BlockSpec((N // 10, D), lambda i: (i, 0)),
        pl.BlockSpec((D, D), lambda i: (0, 0)),
    ],
    out_specs=pl.BlockSpec((N // 10, D), lambda i: (i, 0)),
    out_shape=jax.ShapeDtypeStruct((N, D), jnp.float32),
)


def _add_body(p_ref, o_ref):
    o_ref[...] = p_ref[0] + p_ref[1]


_add_partials = pl.pallas_call(
    _add_body,
    grid=(10,),
    in_specs=[pl.BlockSpec((NC, N // 10, D), lambda i: (0, i, 0))],
    out_specs=pl.BlockSpec((N // 10, D), lambda i: (i, 0)),
    out_shape=jax.ShapeDtypeStruct((N, D), jnp.float32),
)


def kernel(x, adj_indices, adj_values, W):
    pre_sup = _matmul(x, W)
    pad = EP - E
    cols = jnp.pad(adj_indices[1], (0, pad)).reshape(TOTCH, CH, G)
    rows = jnp.pad(adj_indices[0], (0, pad)).reshape(TOTCH, CH, G)
    vals = jnp.pad(adj_values, (0, pad)).reshape(TOTCH, CH, G)
    partials = _spmm_sc(pre_sup, cols, rows, vals)
    return _add_partials(partials)


# speedup vs baseline: 1.2532x; 1.0176x over previous
import functools

import jax
import jax.numpy as jnp
from jax import lax
from jax.experimental import pallas as pl
from jax.experimental.pallas import tpu as pltpu
from jax.experimental.pallas import tpu_sc as plsc

N = 10000
E = 320000
D = 128

NC = 2
NS = 16
NW = NC * NS
G = 112
H = G // 2
CH = 6
CPW = 30
A_CH = 8
B_CH = CPW - A_CH
TOTCH = NS * CPW + 1
EP = TOTCH * CH * G

ROW_STRIDE = 624
ROW_SPAN = 640

_mesh = plsc.VectorSubcoreMesh(core_axis_name="c", subcore_axis_name="s")


@functools.partial(
    pl.kernel,
    out_type=jax.ShapeDtypeStruct((NC, N, D), jnp.float32),
    mesh=_mesh,
    scratch_types=[
        pltpu.VMEM((2, CH, G), jnp.int32),
        pltpu.VMEM((2, CH, G), jnp.int32),
        pltpu.VMEM((2, CH, G), jnp.float32),
        pltpu.VMEM((G, D), jnp.float32),
        pltpu.VMEM((G, D), jnp.float32),
        pltpu.VMEM((G, D), jnp.float32),
        pltpu.VMEM_SHARED((N, D), jnp.float32),
        pltpu.SemaphoreType.DMA,
        pltpu.SemaphoreType.DMA,
        pltpu.SemaphoreType.DMA,
        pltpu.SemaphoreType.DMA,
        pltpu.SemaphoreType.DMA,
        pltpu.SemaphoreType.DMA,
        pltpu.SemaphoreType.DMA,
        pltpu.SemaphoreType.DMA,
        pltpu.SemaphoreType.DMA,
        pltpu.SemaphoreType.DMA,
    ],
)
def _spmm_sc(pre_hbm, cols_hbm, rows_hbm, vals_hbm, out_hbm,
             colsb, rowsb, valsb, gb0, gb1, gb2, acc,
             sg0a, sg0b, sg1a, sg1b, sg2a, sg2b, ss0, ss1, ss2, si):
    c = lax.axis_index("c")
    s = lax.axis_index("s")
    wid = s * NC + c
    bufs = (gb0, gb1, gb2)
    semg = ((sg0a, sg0b), (sg1a, sg1b), (sg2a, sg2b))
    sems = (ss0, ss1, ss2)

    zero = jnp.zeros((16,), jnp.float32)

    def _zrow(r, carry):
        for j in range(D // 16):
            gb0[r, pl.ds(16 * j, 16)] = zero
        return carry

    lax.fori_loop(0, 80, _zrow, 0)
    for k in range(ROW_SPAN // 80):
        pltpu.sync_copy(
            gb0.at[pl.ds(0, 80)],
            acc.at[pl.ds(s * ROW_STRIDE + k * 80, 80)],
        )
    plsc.subcore_barrier()

    def _fetch_idx(chunk, slot):
        pltpu.async_copy(cols_hbm.at[chunk], colsb.at[slot], si)
        pltpu.async_copy(rows_hbm.at[chunk], rowsb.at[slot], si)
        pltpu.async_copy(vals_hbm.at[chunk], valsb.at[slot], si)

    def _wait_idx():
        pltpu.make_async_copy(cols_hbm.at[0], colsb.at[0], si).wait()
        pltpu.make_async_copy(rows_hbm.at[0], rowsb.at[0], si).wait()
        pltpu.make_async_copy(vals_hbm.at[0], valsb.at[0], si).wait()

    def _start_gather(buf, sempair, slot, pos):
        idx = colsb.at[slot, pos]
        pltpu.async_copy(pre_hbm.at[idx.at[pl.ds(0, H)]],
                         buf.at[pl.ds(0, H)], sempair[0])
        pltpu.async_copy(pre_hbm.at[idx.at[pl.ds(H, H)]],
                         buf.at[pl.ds(H, H)], sempair[1])

    def _wait_gather(buf, sempair):
        pltpu.make_async_copy(pre_hbm.at[pl.ds(0, H)],
                              buf.at[pl.ds(0, H)], sempair[0]).wait()
        pltpu.make_async_copy(pre_hbm.at[pl.ds(0, H)],
                              buf.at[pl.ds(H, H)], sempair[1]).wait()

    def _start_scatter(buf, sem, slot, pos):
        pltpu.async_copy(buf, acc.at[rowsb.at[slot, pos]], sem, add=True)

    def _wait_scatter(buf, sem):
        pltpu.make_async_copy(buf, acc.at[pl.ds(0, G)], sem).wait()

    def _scale(buf, slot, pos):
        def _eblock(eb, carry):
            vvec = valsb[slot, pos, pl.ds(eb * 16, 16)]
            for l in range(16):
                v = vvec[l]
                e = eb * 16 + l
                for j in range(D // 16):
                    sl = pl.ds(16 * j, 16)
                    buf[e, sl] = buf[e, sl] * v
            return carry

        lax.fori_loop(0, G // 16, _eblock, 0)

    def _body(g, k, slot, next_slot, first_chunk=False, last_chunk=False):
        k3 = k % 3
        j3 = (k + 1) % 3
        _wait_gather(bufs[k3], semg[k3])
        if not (first_chunk and k < 2):
            _wait_scatter(bufs[j3], sems[j3])
        if k == CH - 1:
            if not last_chunk:
                _wait_idx()
                _start_gather(bufs[j3], semg[j3], next_slot, 0)
        else:
            _start_gather(bufs[j3], semg[j3], slot, k + 1)
        _scale(bufs[k3], slot, k)
        _start_scatter(bufs[k3], sems[k3], slot, k)

    base = s * CPW + c * B_CH
    ncw = jnp.where(c == 0, B_CH, A_CH)

    _fetch_idx(base, 0)
    _wait_idx()
    _start_gather(gb0, semg[0], 0, 0)
    for k in range(2):
        _body(k, k, 0, 1, first_chunk=True)
    _fetch_idx(base + 1, 1)
    for k in range(2, CH):
        _body(k, k, 0, 1)

    def _super(ci, carry):
        slot = lax.rem(ci, 2)
        next_slot = lax.rem(ci + 1, 2)
        for k in range(2):
            _body(CH * ci + k, k, slot, next_slot)
        _fetch_idx(base + ci + 1, next_slot)
        for k in range(2, CH):
            _body(CH * ci + k, k, slot, next_slot)
        return carry

    lax.fori_loop(1, ncw, _super, 0)

    _wait_scatter(bufs[(CH - 2) % 3], sems[(CH - 2) % 3])
    _wait_scatter(bufs[(CH - 1) % 3], sems[(CH - 1) % 3])
    _wait_gather(bufs[0], semg[0])
    plsc.subcore_barrier()

    pltpu.sync_copy(
        acc.at[pl.ds(s * ROW_STRIDE, ROW_SPAN)],
        out_hbm.at[c, pl.ds(s * ROW_STRIDE, ROW_SPAN)],
    )


def _mm_body(x_ref, w_ref, o_ref):
    o_ref[...] = jnp.dot(x_ref[...], w_ref[...],
                         preferred_element_type=jnp.float32)


_matmul = pl.pallas_call(
    _mm_body,
    grid=(10,),
    in_specs=[
        pl.BlockSpec((N // 10, D), lambda i: (i, 0)),
        pl.BlockSpec((D, D), lambda i: (0, 0)),
    ],
    out_specs=pl.BlockSpec((N // 10, D), lambda i: (i, 0)),
    out_shape=jax.ShapeDtypeStruct((N, D), jnp.float32),
)


def _add_body(p_ref, o_ref):
    o_ref[...] = p_ref[0] + p_ref[1]


_add_partials = pl.pallas_call(
    _add_body,
    grid=(10,),
    in_specs=[pl.BlockSpec((NC, N // 10, D), lambda i: (0, i, 0))],
    out_specs=pl.BlockSpec((N // 10, D), lambda i: (i, 0)),
    out_shape=jax.ShapeDtypeStruct((N, D), jnp.float32),
)


def kernel(x, adj_indices, adj_values, W):
    pre_sup = _matmul(x, W)
    pad = EP - E
    cols = jnp.pad(adj_indices[1], (0, pad)).reshape(TOTCH, CH, G)
    rows = jnp.pad(adj_indices[0], (0, pad)).reshape(TOTCH, CH, G)
    vals = jnp.pad(adj_values, (0, pad)).reshape(TOTCH, CH, G)
    partials = _spmm_sc(pre_sup, cols, rows, vals)
    return _add_partials(partials)
